# baseline (speedup 1.0000x reference)
import functools

import jax
import jax.numpy as jnp
from jax import lax
from jax.experimental import pallas as pl
from jax.experimental.pallas import tpu as pltpu
from jax.experimental.pallas import tpu_sc as plsc

TRAIN = 100000
SPLIT = 50048
TAILOFF = 99968
HLEN = (SPLIT, TAILOFF - SPLIT)
BATCH = 16384
BITS = 64
LANES = 16
NC = 2
NS = 16
NW = NC * NS
CPW = BITS // NW
QCH = 4096
NQ = BATCH // QCH
UNROLL = 8
NTASK = CPW * 2

_mesh = plsc.VectorSubcoreMesh(core_axis_name="c", subcore_axis_name="s")


@functools.partial(
    pl.kernel,
    mesh=_mesh,
    compiler_params=pltpu.CompilerParams(needs_layout_passes=False),
    out_type=jax.ShapeDtypeStruct((NW, LANES), jnp.float32),
    scratch_types=[
        pltpu.VMEM((2 * SPLIT,), jnp.float32),
        pltpu.VMEM((BATCH,), jnp.float32),
        pltpu.VMEM((2, QCH), jnp.int32),
        pltpu.VMEM((LANES,), jnp.float32),
        pltpu.SemaphoreType.DMA,
        pltpu.SemaphoreType.DMA,
        pltpu.SemaphoreType.DMA,
        pltpu.SemaphoreType.DMA,
        pltpu.SemaphoreType.DMA,
    ],
)
def _mse_partials(xT_hbm, idx_hbm, HT_hbm, tailT_hbm, out_hbm,
                  hc_v, xc_v, idxq_v, acc_v, sem_h0, sem_h1, sem_x, sem_i0,
                  sem_i1):
    wid = lax.axis_index("s") * NC + lax.axis_index("c")
    c0 = wid * CPW
    sem_h = (sem_h0, sem_h1)
    sem_i = (sem_i0, sem_i1)

    def hc_dma(t):
        k, h = t // 2, t % 2
        cps = [pltpu.async_copy(
            HT_hbm.at[:, pl.ds(h * SPLIT, HLEN[h])].at[c0 + k],
            hc_v.at[pl.ds((t % 2) * SPLIT, HLEN[h])],
            sem_h[t % 2])]
        if h == 1:
            cps.append(pltpu.async_copy(
                tailT_hbm.at[c0 + k],
                hc_v.at[pl.ds((t % 2) * SPLIT + HLEN[1], 128)],
                sem_h[t % 2]))
        return cps

    def idx_dma(p):
        q = p % NQ
        return pltpu.async_copy(
            idx_hbm.at[pl.ds(q * QCH, QCH)], idxq_v.at[p % 2], sem_i[p % 2])

    hc_cp = {0: hc_dma(0), 1: hc_dma(1)}
    x_cp = pltpu.async_copy(xT_hbm.at[c0], xc_v, sem_x)
    idx_cp = {0: idx_dma(0)}

    acc = (jnp.zeros((LANES,), jnp.float32),) * UNROLL
    for t in range(NTASK):
        k, h = t // 2, t % 2
        if t == 2:
            x_cp = pltpu.async_copy(xT_hbm.at[c0 + 1], xc_v, sem_x)
        for cp in hc_cp.pop(t % 2):
            cp.wait()
        if h == 0:
            x_cp.wait()
        for q in range(NQ):
            p = t * NQ + q
            if p + 1 < NTASK * NQ:
                idx_cp[(p + 1) % 2] = idx_dma(p + 1)
            idx_cp.pop(p % 2).wait()

            def chunk_body(i, accs, q=q, h=h, b=t % 2, pb=p % 2):
                idx16 = idxq_v[pb, pl.ds(i * LANES, LANES)]
                x16 = xc_v[pl.ds(q * QCH + i * LANES, LANES)]
                if h == 0:
                    mask = idx16 < SPLIT
                    within = idx16
                else:
                    mask = idx16 >= SPLIT
                    within = idx16 - SPLIT
                h16 = plsc.load_gather(
                    hc_v.at[pl.ds(b * SPLIT, SPLIT)], [within], mask=mask)
                d = x16 - h16
                sq = jnp.where(mask, d * d, jnp.float32(0.0))
                return accs[1:] + (accs[0] + sq,)

            acc = plsc.parallel_loop(
                0, QCH // LANES, unroll=UNROLL, carry=acc)(chunk_body)
        if t + 2 < NTASK:
            hc_cp[t % 2] = hc_dma(t + 2)

    total = acc[0]
    for u in range(1, UNROLL):
        total = total + acc[u]
    acc_v[...] = total
    pltpu.sync_copy(acc_v, out_hbm.at[wid])


def kernel(x, y, idx, H):
    HT = H.T
    tailT = jnp.pad(HT[:, TAILOFF:], ((0, 0), (0, 128 - (TRAIN - TAILOFF))))
    partials = _mse_partials(x.T, idx.astype(jnp.int32), HT, tailT)
    return jnp.sum(partials) / jnp.float32(BATCH * BITS)


# speedup vs baseline: 1.0047x; 1.0047x over previous
import functools

import jax
import jax.numpy as jnp
from jax import lax
from jax.experimental import pallas as pl
from jax.experimental.pallas import tpu as pltpu
from jax.experimental.pallas import tpu_sc as plsc

TRAIN = 100000
SPLIT = 50048
TAILOFF = 99968
HLEN = (SPLIT, TAILOFF - SPLIT)
BATCH = 16384
BITS = 64
LANES = 16
NC = 2
NS = 16
NW = NC * NS
CPW = BITS // NW
QCH = 4096
NQ = BATCH // QCH
UNROLL = 8
NTASK = CPW * 2

_mesh = plsc.VectorSubcoreMesh(core_axis_name="c", subcore_axis_name="s")


@functools.partial(
    pl.kernel,
    mesh=_mesh,
    compiler_params=pltpu.CompilerParams(needs_layout_passes=False),
    out_type=jax.ShapeDtypeStruct((NW, LANES), jnp.float32),
    scratch_types=[
        pltpu.VMEM((2 * SPLIT,), jnp.float32),
        pltpu.VMEM((BATCH,), jnp.float32),
        pltpu.VMEM((2, QCH), jnp.int32),
        pltpu.VMEM((LANES,), jnp.float32),
        pltpu.SemaphoreType.DMA,
        pltpu.SemaphoreType.DMA,
        pltpu.SemaphoreType.DMA,
        pltpu.SemaphoreType.DMA,
        pltpu.SemaphoreType.DMA,
    ],
)
def _mse_partials(xT_hbm, idx_hbm, HT_hbm, tailT_hbm, out_hbm,
                  hc_v, xc_v, idxq_v, acc_v, sem_h0, sem_h1, sem_x, sem_i0,
                  sem_i1):
    wid = lax.axis_index("s") * NC + lax.axis_index("c")
    c0 = wid * CPW
    sem_h = (sem_h0, sem_h1)
    sem_i = (sem_i0, sem_i1)

    def hc_dma(t):
        k, h = t // 2, t % 2
        cps = [pltpu.async_copy(
            HT_hbm.at[:, pl.ds(h * SPLIT, HLEN[h])].at[c0 + k],
            hc_v.at[pl.ds((t % 2) * SPLIT, HLEN[h])],
            sem_h[t % 2])]
        if h == 1:
            cps.append(pltpu.async_copy(
                tailT_hbm.at[c0 + k],
                hc_v.at[pl.ds((t % 2) * SPLIT + HLEN[1], 128)],
                sem_h[t % 2]))
        return cps

    def idx_dma(p):
        q = p % NQ
        return pltpu.async_copy(
            idx_hbm.at[pl.ds(q * QCH, QCH)], idxq_v.at[p % 2], sem_i[p % 2])

    hc_cp = {0: hc_dma(0)}
    x_cp = pltpu.async_copy(xT_hbm.at[c0], xc_v, sem_x)
    idx_cp = {0: idx_dma(0)}

    acc = (jnp.zeros((LANES,), jnp.float32),) * UNROLL
    for t in range(NTASK):
        k, h = t // 2, t % 2
        if t == 2:
            x_cp = pltpu.async_copy(xT_hbm.at[c0 + 1], xc_v, sem_x)
        for cp in hc_cp.pop(t % 2):
            cp.wait()
        if t + 1 < NTASK:
            hc_cp[(t + 1) % 2] = hc_dma(t + 1)
        if h == 0:
            x_cp.wait()
        for q in range(NQ):
            p = t * NQ + q
            if p + 1 < NTASK * NQ:
                idx_cp[(p + 1) % 2] = idx_dma(p + 1)
            idx_cp.pop(p % 2).wait()

            def chunk_body(i, accs, q=q, h=h, b=t % 2, pb=p % 2):
                idx16 = idxq_v[pb, pl.ds(i * LANES, LANES)]
                x16 = xc_v[pl.ds(q * QCH + i * LANES, LANES)]
                if h == 0:
                    mask = idx16 < SPLIT
                    within = idx16
                else:
                    mask = idx16 >= SPLIT
                    within = idx16 - SPLIT
                h16 = plsc.load_gather(
                    hc_v.at[pl.ds(b * SPLIT, SPLIT)], [within], mask=mask)
                d = x16 - h16
                sq = jnp.where(mask, d * d, jnp.float32(0.0))
                return accs[1:] + (accs[0] + sq,)

            acc = plsc.parallel_loop(
                0, QCH // LANES, unroll=UNROLL, carry=acc)(chunk_body)

    total = acc[0]
    for u in range(1, UNROLL):
        total = total + acc[u]
    acc_v[...] = total
    pltpu.sync_copy(acc_v, out_hbm.at[wid])


def kernel(x, y, idx, H):
    HT = H.T
    tailT = jnp.pad(HT[:, TAILOFF:], ((0, 0), (0, 128 - (TRAIN - TAILOFF))))
    partials = _mse_partials(x.T, idx.astype(jnp.int32), HT, tailT)
    return jnp.sum(partials) / jnp.float32(BATCH * BITS)


# speedup vs baseline: 1.1864x; 1.1809x over previous
import functools

import jax
import jax.numpy as jnp
from jax import lax
from jax.experimental import pallas as pl
from jax.experimental.pallas import tpu as pltpu
from jax.experimental.pallas import tpu_sc as plsc

TRAIN = 100000
BATCH = 16384
BITS = 64
LANES = 16
NC = 2
NS = 16
NW = NC * NS
CPW = BITS // NW
QCH = 4096
NQ = BATCH // QCH
UNROLL = 8

_mesh = plsc.VectorSubcoreMesh(core_axis_name="c", subcore_axis_name="s")


@functools.partial(
    pl.kernel,
    mesh=_mesh,
    compiler_params=pltpu.CompilerParams(needs_layout_passes=False),
    out_type=jax.ShapeDtypeStruct((NW, LANES), jnp.float32),
    scratch_types=[
        pltpu.VMEM((TRAIN,), jnp.float32),
        pltpu.VMEM((BATCH,), jnp.int32),
        pltpu.VMEM((2, QCH), jnp.float32),
        pltpu.VMEM((LANES,), jnp.float32),
        pltpu.SemaphoreType.DMA,
        pltpu.SemaphoreType.DMA,
        pltpu.SemaphoreType.DMA,
        pltpu.SemaphoreType.DMA,
    ],
)
def _mse_partials(xT_hbm, idx_hbm, HT_hbm, out_hbm,
                  hc_v, idx_v, xq_v, acc_v, sem_h, sem_x0, sem_x1, sem_i):
    wid = lax.axis_index("s") * NC + lax.axis_index("c")
    sem_x = (sem_x0, sem_x1)

    idx_cp = pltpu.async_copy(idx_hbm, idx_v, sem_i)
    acc = (jnp.zeros((LANES,), jnp.float32),) * UNROLL
    for k in range(CPW):
        c = wid * CPW + k
        hc_cp = pltpu.async_copy(HT_hbm.at[c], hc_v, sem_h)
        pltpu.async_copy(xT_hbm.at[c, pl.ds(0, QCH)], xq_v.at[0], sem_x[0])
        hc_cp.wait()
        if k == 0:
            idx_cp.wait()
        for q in range(NQ):
            if q + 1 < NQ:
                pltpu.async_copy(
                    xT_hbm.at[c, pl.ds((q + 1) * QCH, QCH)],
                    xq_v.at[(q + 1) % 2], sem_x[(q + 1) % 2])
            pltpu.make_async_copy(
                xT_hbm.at[c, pl.ds(0, QCH)], xq_v.at[q % 2], sem_x[q % 2]
            ).wait()

            def chunk_body(i, accs, q=q):
                idx16 = idx_v[pl.ds(q * QCH + i * LANES, LANES)]
                h16 = plsc.load_gather(hc_v, [idx16])
                x16 = xq_v[q % 2, pl.ds(i * LANES, LANES)]
                d = x16 - h16
                return accs[1:] + (accs[0] + d * d,)

            acc = plsc.parallel_loop(
                0, QCH // LANES, unroll=UNROLL, carry=acc)(chunk_body)

    total = acc[0]
    for u in range(1, UNROLL):
        total = total + acc[u]
    acc_v[...] = total
    pltpu.sync_copy(acc_v, out_hbm.at[wid])


def kernel(x, y, idx, H):
    partials = _mse_partials(x.T, idx.astype(jnp.int32), H.T)
    return jnp.sum(partials) / jnp.float32(BATCH * BITS)
